# Initial kernel scaffold; baseline (speedup 1.0000x reference)
import jax
import jax.numpy as jnp
from jax.experimental import pallas as pl
from jax.experimental.pallas import tpu as pltpu

NQ = 8
K = 1024
D = 256
B = 16
S = 1536


def _rvq_body(z_ref, cb_ref, q_ref, idx_ref, loss_ref, perp_ref,
              counts_ref, loss_acc_ref, cn_ref):
    b = pl.program_id(0)
    nsteps = pl.num_programs(0)
    first = b == 0
    last = b == nsteps - 1

    @pl.when(first)
    def _init():
        counts_ref[...] = jnp.zeros_like(counts_ref)
        loss_acc_ref[...] = jnp.zeros_like(loss_acc_ref)
        cb = cb_ref[...]
        cn_ref[...] = 0.5 * jnp.sum(cb * cb, axis=2)

    r = z_ref[0]
    quant = jnp.zeros_like(r)
    for q in range(NQ):
        c = cb_ref[q]
        scores = jax.lax.dot_general(
            c, r, (((1,), (0,)), ((), ())),
            preferred_element_type=jnp.float32,
            precision=jax.lax.Precision.HIGHEST) - cn_ref[q][:, None]
        m = jnp.max(scores, axis=0, keepdims=True)
        iota = jax.lax.broadcasted_iota(jnp.int32, (K, S), 0)
        hit = scores == m
        idx = jnp.min(jnp.where(hit, iota, K), axis=0)
        onehot = (iota == idx[None, :]).astype(jnp.float32)
        sel = jax.lax.dot_general(
            c, onehot, (((0,), (0,)), ((), ())),
            preferred_element_type=jnp.float32,
            precision=jax.lax.Precision.HIGHEST)
        quant = quant + sel
        r = r - sel
        idx_ref[0, q, :] = idx
        counts_ref[q, :] += jnp.sum(onehot, axis=1)

    q_ref[0] = quant
    loss_acc_ref[0, 0] += jnp.sum(r * r)

    @pl.when(last)
    def _epilogue():
        counts = counts_ref[...]
        total = jnp.clip(jnp.sum(counts, axis=1, keepdims=True), 1.0, None)
        probs = counts / total
        ent = -jnp.sum(probs * jnp.log(probs + 1e-10), axis=1)
        perp_ref[0, 0] = jnp.mean(jnp.exp(ent))
        loss_ref[0, 0] = loss_acc_ref[0, 0] / (B * S * D)


@jax.jit
def kernel(z, codebooks):
    quant, idx, loss, perp = pl.pallas_call(
        _rvq_body,
        grid=(B,),
        in_specs=[
            pl.BlockSpec((1, D, S), lambda b: (b, 0, 0)),
            pl.BlockSpec((NQ, K, D), lambda b: (0, 0, 0)),
        ],
        out_specs=[
            pl.BlockSpec((1, D, S), lambda b: (b, 0, 0)),
            pl.BlockSpec((1, NQ, S), lambda b: (b, 0, 0)),
            pl.BlockSpec((1, 1), lambda b: (0, 0)),
            pl.BlockSpec((1, 1), lambda b: (0, 0)),
        ],
        out_shape=[
            jax.ShapeDtypeStruct((B, D, S), jnp.float32),
            jax.ShapeDtypeStruct((B, NQ, S), jnp.int32),
            jax.ShapeDtypeStruct((1, 1), jnp.float32),
            jax.ShapeDtypeStruct((1, 1), jnp.float32),
        ],
        scratch_shapes=[
            pltpu.VMEM((NQ, K), jnp.float32),
            pltpu.VMEM((1, 1), jnp.float32),
            pltpu.VMEM((NQ, K), jnp.float32),
        ],
    )(z, codebooks)
    return (quant,
            jnp.transpose(idx, (0, 2, 1)),
            loss[0, 0],
            perp[0, 0])


# speedup vs baseline: 1.2798x; 1.2798x over previous
import jax
import jax.numpy as jnp
from jax.experimental import pallas as pl
from jax.experimental.pallas import tpu as pltpu

NQ = 8
K = 1024
D = 256
B = 16
S = 1536


def _rvq_body(z_ref, cb_ref, q_ref, idx_ref, loss_ref, perp_ref,
              counts_ref, loss_acc_ref, cn_ref):
    b = pl.program_id(0)
    nsteps = pl.num_programs(0)
    first = b == 0
    last = b == nsteps - 1

    @pl.when(first)
    def _init():
        counts_ref[...] = jnp.zeros_like(counts_ref)
        loss_acc_ref[...] = jnp.zeros_like(loss_acc_ref)
        cb = cb_ref[...]
        cn_ref[...] = 0.5 * jnp.sum(cb * cb, axis=2)

    r = z_ref[0]
    quant = jnp.zeros_like(r)
    for q in range(NQ):
        c = cb_ref[q]
        scores = jax.lax.dot_general(
            c, r, (((1,), (0,)), ((), ())),
            preferred_element_type=jnp.float32,
            precision=jax.lax.Precision.DEFAULT) - cn_ref[q][:, None]
        m = jnp.max(scores, axis=0, keepdims=True)
        iota = jax.lax.broadcasted_iota(jnp.int32, (K, S), 0)
        hit = scores == m
        idx = jnp.min(jnp.where(hit, iota, K), axis=0)
        onehot = (iota == idx[None, :]).astype(jnp.float32)
        sel = jax.lax.dot_general(
            c, onehot, (((0,), (0,)), ((), ())),
            preferred_element_type=jnp.float32,
            precision=jax.lax.Precision.HIGHEST)
        quant = quant + sel
        r = r - sel
        idx_ref[0, q, :] = idx
        counts_ref[q, :] += jnp.sum(onehot, axis=1)

    q_ref[0] = quant
    loss_acc_ref[...] = loss_acc_ref[...] + jnp.sum(r * r)

    @pl.when(last)
    def _epilogue():
        counts = counts_ref[...]
        total = jnp.clip(jnp.sum(counts, axis=1, keepdims=True), 1.0, None)
        probs = counts / total
        ent = -jnp.sum(probs * jnp.log(probs + 1e-10), axis=1, keepdims=True)
        perp_ref[...] = jnp.mean(jnp.exp(ent)).reshape(1, 1)
        loss_ref[...] = loss_acc_ref[...] / (B * S * D)


@jax.jit
def kernel(z, codebooks):
    quant, idx, loss, perp = pl.pallas_call(
        _rvq_body,
        grid=(B,),
        in_specs=[
            pl.BlockSpec((1, D, S), lambda b: (b, 0, 0)),
            pl.BlockSpec((NQ, K, D), lambda b: (0, 0, 0)),
        ],
        out_specs=[
            pl.BlockSpec((1, D, S), lambda b: (b, 0, 0)),
            pl.BlockSpec((1, NQ, S), lambda b: (b, 0, 0)),
            pl.BlockSpec((1, 1), lambda b: (0, 0)),
            pl.BlockSpec((1, 1), lambda b: (0, 0)),
        ],
        out_shape=[
            jax.ShapeDtypeStruct((B, D, S), jnp.float32),
            jax.ShapeDtypeStruct((B, NQ, S), jnp.int32),
            jax.ShapeDtypeStruct((1, 1), jnp.float32),
            jax.ShapeDtypeStruct((1, 1), jnp.float32),
        ],
        scratch_shapes=[
            pltpu.VMEM((NQ, K), jnp.float32),
            pltpu.VMEM((1, 1), jnp.float32),
            pltpu.VMEM((NQ, K), jnp.float32),
        ],
    )(z, codebooks)
    return (quant,
            jnp.transpose(idx, (0, 2, 1)),
            loss[0, 0],
            perp[0, 0])


# speedup vs baseline: 2.0806x; 1.6257x over previous
import jax
import jax.numpy as jnp
from jax.experimental import pallas as pl
from jax.experimental.pallas import tpu as pltpu

NQ = 8
K = 1024
D = 256
B = 16
S = 1536


def _rvq_body(z_ref, cb_ref, q_ref, idx_ref, loss_ref, perp_ref,
              counts_ref, loss_acc_ref, cn_ref, mid_ref, low_ref):
    b = pl.program_id(0)
    nsteps = pl.num_programs(0)
    first = b == 0
    last = b == nsteps - 1

    @pl.when(first)
    def _init():
        counts_ref[...] = jnp.zeros_like(counts_ref)
        loss_acc_ref[...] = jnp.zeros_like(loss_acc_ref)
        cb = cb_ref[...]
        cn_ref[...] = 0.5 * jnp.sum(cb * cb, axis=2)
        hi = cb.astype(jnp.bfloat16).astype(jnp.float32)
        mid_f = (cb - hi).astype(jnp.bfloat16).astype(jnp.float32)
        mid_ref[...] = mid_f
        low_ref[...] = cb - hi - mid_f

    r = z_ref[0]
    quant = jnp.zeros_like(r)
    iota = jax.lax.broadcasted_iota(jnp.int32, (K, S), 0)
    for q in range(NQ):
        c = cb_ref[q]
        scores = jax.lax.dot_general(
            c, r, (((1,), (0,)), ((), ())),
            preferred_element_type=jnp.float32,
            precision=jax.lax.Precision.DEFAULT) - cn_ref[q][:, None]
        idx = jnp.argmax(scores, axis=0)
        onehot = jnp.where(iota == idx[None, :], 1.0, 0.0)
        sel = None
        for plane in (c, mid_ref[q], low_ref[q]):
            part = jax.lax.dot_general(
                plane, onehot, (((0,), (0,)), ((), ())),
                preferred_element_type=jnp.float32,
                precision=jax.lax.Precision.DEFAULT)
            sel = part if sel is None else sel + part
        quant = quant + sel
        r = r - sel
        idx_ref[0, q, :] = idx
        counts_ref[q, :] += jnp.sum(onehot, axis=1)

    q_ref[0] = quant
    loss_acc_ref[...] = loss_acc_ref[...] + jnp.sum(r * r)

    @pl.when(last)
    def _epilogue():
        counts = counts_ref[...]
        total = jnp.clip(jnp.sum(counts, axis=1, keepdims=True), 1.0, None)
        probs = counts / total
        ent = -jnp.sum(probs * jnp.log(probs + 1e-10), axis=1, keepdims=True)
        perp_ref[...] = jnp.mean(jnp.exp(ent)).reshape(1, 1)
        loss_ref[...] = loss_acc_ref[...] / (B * S * D)


@jax.jit
def kernel(z, codebooks):
    quant, idx, loss, perp = pl.pallas_call(
        _rvq_body,
        grid=(B,),
        in_specs=[
            pl.BlockSpec((1, D, S), lambda b: (b, 0, 0)),
            pl.BlockSpec((NQ, K, D), lambda b: (0, 0, 0)),
        ],
        out_specs=[
            pl.BlockSpec((1, D, S), lambda b: (b, 0, 0)),
            pl.BlockSpec((1, NQ, S), lambda b: (b, 0, 0)),
            pl.BlockSpec((1, 1), lambda b: (0, 0)),
            pl.BlockSpec((1, 1), lambda b: (0, 0)),
        ],
        out_shape=[
            jax.ShapeDtypeStruct((B, D, S), jnp.float32),
            jax.ShapeDtypeStruct((B, NQ, S), jnp.int32),
            jax.ShapeDtypeStruct((1, 1), jnp.float32),
            jax.ShapeDtypeStruct((1, 1), jnp.float32),
        ],
        scratch_shapes=[
            pltpu.VMEM((NQ, K), jnp.float32),
            pltpu.VMEM((1, 1), jnp.float32),
            pltpu.VMEM((NQ, K), jnp.float32),
            pltpu.VMEM((NQ, K, D), jnp.float32),
            pltpu.VMEM((NQ, K, D), jnp.float32),
        ],
    )(z, codebooks)
    return (quant,
            jnp.transpose(idx, (0, 2, 1)),
            loss[0, 0],
            perp[0, 0])


# speedup vs baseline: 3.3946x; 1.6316x over previous
import jax
import jax.numpy as jnp
from jax.experimental import pallas as pl
from jax.experimental.pallas import tpu as pltpu

NQ = 8
K = 1024
D = 256
B = 16
S = 1536


def _rvq_body(z_ref, cb_ref, q_ref, idx_ref, loss_ref, perp_ref,
              counts_ref, loss_acc_ref, cn_ref, mid_ref):
    b = pl.program_id(0)
    nsteps = pl.num_programs(0)
    first = b == 0
    last = b == nsteps - 1

    @pl.when(first)
    def _init():
        counts_ref[...] = jnp.zeros_like(counts_ref)
        loss_acc_ref[...] = jnp.zeros_like(loss_acc_ref)
        cb = cb_ref[...]
        cn_ref[...] = 0.5 * jnp.sum(cb * cb, axis=2)
        hi = cb.astype(jnp.bfloat16).astype(jnp.float32)
        mid_ref[...] = cb - hi

    r = z_ref[0]
    quant = jnp.zeros_like(r)
    iota = jax.lax.broadcasted_iota(jnp.int32, (K, S), 0)
    for q in range(NQ):
        c = cb_ref[q]
        scores = jax.lax.dot_general(
            c, r, (((1,), (0,)), ((), ())),
            preferred_element_type=jnp.float32,
            precision=jax.lax.Precision.DEFAULT) - cn_ref[q][:, None]
        idx = jnp.argmax(scores, axis=0)
        onehot = jnp.where(iota == idx[None, :], 1.0, 0.0)
        sel = None
        for plane in (c, mid_ref[q]):
            part = jax.lax.dot_general(
                plane, onehot, (((0,), (0,)), ((), ())),
                preferred_element_type=jnp.float32,
                precision=jax.lax.Precision.DEFAULT)
            sel = part if sel is None else sel + part
        quant = quant + sel
        r = r - sel
        idx_ref[0, q, :] = idx
        counts_ref[q, :] += jnp.sum(onehot, axis=1)

    q_ref[0] = quant
    loss_acc_ref[...] = loss_acc_ref[...] + jnp.sum(r * r)

    @pl.when(last)
    def _epilogue():
        counts = counts_ref[...]
        total = jnp.clip(jnp.sum(counts, axis=1, keepdims=True), 1.0, None)
        probs = counts / total
        ent = -jnp.sum(probs * jnp.log(probs + 1e-10), axis=1, keepdims=True)
        perp_ref[...] = jnp.mean(jnp.exp(ent)).reshape(1, 1)
        loss_ref[...] = loss_acc_ref[...] / (B * S * D)


@jax.jit
def kernel(z, codebooks):
    quant, idx, loss, perp = pl.pallas_call(
        _rvq_body,
        grid=(B,),
        in_specs=[
            pl.BlockSpec((1, D, S), lambda b: (b, 0, 0)),
            pl.BlockSpec((NQ, K, D), lambda b: (0, 0, 0)),
        ],
        out_specs=[
            pl.BlockSpec((1, D, S), lambda b: (b, 0, 0)),
            pl.BlockSpec((1, NQ, S), lambda b: (b, 0, 0)),
            pl.BlockSpec((1, 1), lambda b: (0, 0)),
            pl.BlockSpec((1, 1), lambda b: (0, 0)),
        ],
        out_shape=[
            jax.ShapeDtypeStruct((B, D, S), jnp.float32),
            jax.ShapeDtypeStruct((B, NQ, S), jnp.int32),
            jax.ShapeDtypeStruct((1, 1), jnp.float32),
            jax.ShapeDtypeStruct((1, 1), jnp.float32),
        ],
        scratch_shapes=[
            pltpu.VMEM((NQ, K), jnp.float32),
            pltpu.VMEM((1, 1), jnp.float32),
            pltpu.VMEM((NQ, K), jnp.float32),
            pltpu.VMEM((NQ, K, D), jnp.float32),
        ],
    )(z, codebooks)
    return (quant,
            jnp.transpose(idx, (0, 2, 1)),
            loss[0, 0],
            perp[0, 0])
